# Initial kernel scaffold; baseline (speedup 1.0000x reference)
import jax
import jax.numpy as jnp
from jax.experimental import pallas as pl


def kernel(input_ids, embedding):
    raise NotImplementedError("write your pallas kernel here")


# speedup vs baseline: 3.4205x; 3.4205x over previous
import jax
import jax.numpy as jnp
from jax.experimental import pallas as pl


def _copy_block(emb_ref, out_ref):
    out_ref[...] = emb_ref[...]


def kernel(input_ids, embedding):
    L = input_ids.shape[1]
    D = embedding.shape[1]
    BLOCK = 512
    out = pl.pallas_call(
        _copy_block,
        grid=(L // BLOCK,),
        in_specs=[pl.BlockSpec((BLOCK, D), lambda i: (i, 0))],
        out_specs=pl.BlockSpec((BLOCK, D), lambda i: (i, 0)),
        out_shape=jax.ShapeDtypeStruct((L, D), embedding.dtype),
    )(embedding)
    return out[None]
